# baseline (speedup 1.0000x reference)
import functools

import jax
import jax.numpy as jnp
from jax import lax
from jax.experimental import pallas as pl
from jax.experimental.pallas import tpu as pltpu
from jax.experimental.pallas import tpu_sc as plsc

N = 10000
E = 320000
D = 128
H = 256
HH = 128
NSUB = 16
CHUNK = 128
NCHUNK = 159
EPW = NCHUNK * CHUNK
E_PAD = EPW * NSUB
ACC_ROWS = 10016
ROWS_PER_SUB = N // NSUB


def _sc_body2(h2_hbm, h3_hbm, src_hbm, dst_hbm, out_hbm,
              srcbuf_a, dstbuf_a, idxbuf_a, rows_a,
              srcbuf_b, dstbuf_b, idxbuf_b, rows_b,
              srcbuf_c, dstbuf_c, idxbuf_c, rows_c,
              acc, sem_i, sem_a, sem_b, sem_c):
  c = lax.axis_index("c")
  s = lax.axis_index("s")
  row0 = s * ROWS_PER_SUB
  init = pltpu.async_copy(h3_hbm.at[pl.ds(row0, ROWS_PER_SUB), c],
                          acc.at[pl.ds(row0, ROWS_PER_SUB)], sem_i)
  init.wait()
  plsc.subcore_barrier()

  ebase = s * EPW

  def prep(i, srcbuf, dstbuf, idxbuf, rows, sem):
    off = ebase + i * CHUNK
    pltpu.sync_copy(src_hbm.at[pl.ds(off, CHUNK)], srcbuf)
    pltpu.sync_copy(dst_hbm.at[pl.ds(off, CHUNK)], dstbuf)
    for j in range(CHUNK // 16):
      sl = pl.ds(j * 16, 16)
      idxbuf[sl] = srcbuf[sl] * 2 + c
    pltpu.async_copy(h2_hbm.at[idxbuf], rows, sem)

  def finish(idxbuf, dstbuf, rows, sem):
    pltpu.make_async_copy(h2_hbm.at[idxbuf], rows, sem).wait()
    pltpu.sync_copy(rows, acc.at[dstbuf], add=True)

  def prep_a(i):
    prep(i, srcbuf_a, dstbuf_a, idxbuf_a, rows_a, sem_a)

  def prep_b(i):
    prep(i, srcbuf_b, dstbuf_b, idxbuf_b, rows_b, sem_b)

  def prep_c(i):
    prep(i, srcbuf_c, dstbuf_c, idxbuf_c, rows_c, sem_c)

  def fin_a():
    finish(idxbuf_a, dstbuf_a, rows_a, sem_a)

  def fin_b():
    finish(idxbuf_b, dstbuf_b, rows_b, sem_b)

  def fin_c():
    finish(idxbuf_c, dstbuf_c, rows_c, sem_c)

  prep_a(0)
  prep_b(1)

  def triple(jj, carry):
    i = jj * 3
    prep_c(i + 2)
    fin_a()
    prep_a(i + 3)
    fin_b()
    prep_b(i + 4)
    fin_c()
    return carry

  lax.fori_loop(0, NCHUNK // 3 - 2, triple, 0)
  i = NCHUNK - 6
  prep_c(i + 2)
  fin_a()
  prep_a(i + 3)
  fin_b()
  prep_b(i + 4)
  fin_c()
  prep_c(i + 5)
  fin_a()
  fin_b()
  fin_c()

  plsc.subcore_barrier()
  pltpu.sync_copy(acc.at[pl.ds(row0, ROWS_PER_SUB)],
                  out_hbm.at[pl.ds(row0, ROWS_PER_SUB), c])


_sc_scatter = pl.kernel(
    _sc_body2,
    out_type=jax.ShapeDtypeStruct((N, 2, HH), jnp.float32),
    mesh=plsc.VectorSubcoreMesh(core_axis_name="c", subcore_axis_name="s"),
    scratch_types=[
        pltpu.VMEM((CHUNK,), jnp.int32),
        pltpu.VMEM((CHUNK,), jnp.int32),
        pltpu.VMEM((CHUNK,), jnp.int32),
        pltpu.VMEM((CHUNK, HH), jnp.float32),
        pltpu.VMEM((CHUNK,), jnp.int32),
        pltpu.VMEM((CHUNK,), jnp.int32),
        pltpu.VMEM((CHUNK,), jnp.int32),
        pltpu.VMEM((CHUNK, HH), jnp.float32),
        pltpu.VMEM((CHUNK,), jnp.int32),
        pltpu.VMEM((CHUNK,), jnp.int32),
        pltpu.VMEM((CHUNK,), jnp.int32),
        pltpu.VMEM((CHUNK, HH), jnp.float32),
        pltpu.VMEM_SHARED((ACC_ROWS, HH), jnp.float32),
        pltpu.SemaphoreType.DMA,
        pltpu.SemaphoreType.DMA,
        pltpu.SemaphoreType.DMA,
        pltpu.SemaphoreType.DMA,
    ],
)

_BLK = 1000


def _enc_body(x_ref, w_ref, b_ref, o_ref):
  o_ref[...] = (
      jnp.dot(x_ref[...], w_ref[...], preferred_element_type=jnp.float32)
      + b_ref[...])


_enc = pl.pallas_call(
    _enc_body,
    grid=(N // _BLK,),
    in_specs=[
        pl.BlockSpec((_BLK, D), lambda i: (i, 0)),
        pl.BlockSpec((D, H), lambda i: (0, 0)),
        pl.BlockSpec((1, H), lambda i: (0, 0)),
    ],
    out_specs=pl.BlockSpec((_BLK, H), lambda i: (i, 0)),
    out_shape=jax.ShapeDtypeStruct((N, H), jnp.float32),
)


def _mlp_body(s_ref, w1_ref, b1_ref, w2_ref, b2_ref, o_ref, *, elu):
  t = (jnp.dot(s_ref[...], w1_ref[...], preferred_element_type=jnp.float32)
       + b1_ref[...])
  t = jnp.maximum(t, 0.0)
  y = (jnp.dot(t, w2_ref[...], preferred_element_type=jnp.float32)
       + b2_ref[...])
  if elu:
    y = jnp.where(y > 0, y, jnp.exp(jnp.minimum(y, 0.0)) - 1.0)
  o_ref[...] = y


def _make_mlp(elu):
  return pl.pallas_call(
      functools.partial(_mlp_body, elu=elu),
      grid=(N // _BLK,),
      in_specs=[
          pl.BlockSpec((_BLK, H), lambda i: (i, 0)),
          pl.BlockSpec((H, H), lambda i: (0, 0)),
          pl.BlockSpec((1, H), lambda i: (0, 0)),
          pl.BlockSpec((H, H), lambda i: (0, 0)),
          pl.BlockSpec((1, H), lambda i: (0, 0)),
      ],
      out_specs=pl.BlockSpec((_BLK, H), lambda i: (i, 0)),
      out_shape=jax.ShapeDtypeStruct((N, H), jnp.float32),
  )


_mlp_elu = _make_mlp(True)
_mlp_plain = _make_mlp(False)


@jax.jit
def kernel(x, edge_index, W_enc, b_enc, W1_0, b1_0, W2_0, b2_0,
           W1_1, b1_1, W2_1, b2_1):
  src = edge_index[0]
  dst = edge_index[1]
  srcp = jnp.concatenate([src, jnp.zeros((E_PAD - E,), jnp.int32)])
  dstp = jnp.concatenate([dst, jnp.full((E_PAD - E,), N, jnp.int32)])

  def gin(h, W1, b1, W2, b2, mlp):
    s = _sc_scatter(h.reshape(2 * N, HH), h.reshape(N, 2, HH), srcp, dstp)
    return mlp(s.reshape(N, H), W1, b1.reshape(1, H), W2, b2.reshape(1, H))

  h = _enc(x, W_enc, b_enc.reshape(1, H))
  h = gin(h, W1_0, b1_0, W2_0, b2_0, _mlp_elu)
  h = gin(h, W1_1, b1_1, W2_1, b2_1, _mlp_plain)
  return h


# speedup vs baseline: 1.2205x; 1.2205x over previous
import functools

import jax
import jax.numpy as jnp
from jax import lax
from jax.experimental import pallas as pl
from jax.experimental.pallas import tpu as pltpu
from jax.experimental.pallas import tpu_sc as plsc

N = 10000
E = 320000
D = 128
H = 256
HH = 128
NSUB = 16
CHUNK = 128
NCHUNK = 158
EPW = NCHUNK * CHUNK
E_PAD = EPW * NSUB
ACC_ROWS = 10016
ROWS_PER_SUB = N // NSUB


def _sc_body2(h2_hbm, h3_hbm, src_hbm, dst_hbm, out_hbm,
              srcbuf_a, dstbuf_a, idxbuf_a, rows_a,
              srcbuf_b, dstbuf_b, idxbuf_b, rows_b,
              acc, sem_i, sem_a, sem_b):
  c = lax.axis_index("c")
  s = lax.axis_index("s")
  row0 = s * ROWS_PER_SUB
  init = pltpu.async_copy(h3_hbm.at[pl.ds(row0, ROWS_PER_SUB), c],
                          acc.at[pl.ds(row0, ROWS_PER_SUB)], sem_i)
  init.wait()
  plsc.subcore_barrier()

  ebase = s * EPW

  def prep(i, srcbuf, dstbuf, idxbuf, rows, sem):
    off = ebase + i * CHUNK
    pltpu.sync_copy(src_hbm.at[pl.ds(off, CHUNK)], srcbuf)
    pltpu.sync_copy(dst_hbm.at[pl.ds(off, CHUNK)], dstbuf)
    for j in range(CHUNK // 16):
      sl = pl.ds(j * 16, 16)
      idxbuf[sl] = srcbuf[sl] * 2 + c
    pltpu.async_copy(h2_hbm.at[idxbuf], rows, sem)

  def finish(idxbuf, dstbuf, rows, sem):
    pltpu.make_async_copy(h2_hbm.at[idxbuf], rows, sem).wait()
    pltpu.sync_copy(rows, acc.at[dstbuf], add=True)

  prep(0, srcbuf_a, dstbuf_a, idxbuf_a, rows_a, sem_a)

  def pair(jj, carry):
    j0 = jj * 2
    prep(j0 + 1, srcbuf_b, dstbuf_b, idxbuf_b, rows_b, sem_b)
    finish(idxbuf_a, dstbuf_a, rows_a, sem_a)
    prep(j0 + 2, srcbuf_a, dstbuf_a, idxbuf_a, rows_a, sem_a)
    finish(idxbuf_b, dstbuf_b, rows_b, sem_b)
    return carry

  lax.fori_loop(0, NCHUNK // 2 - 1, pair, 0)
  prep(NCHUNK - 1, srcbuf_b, dstbuf_b, idxbuf_b, rows_b, sem_b)
  finish(idxbuf_a, dstbuf_a, rows_a, sem_a)
  finish(idxbuf_b, dstbuf_b, rows_b, sem_b)

  plsc.subcore_barrier()
  pltpu.sync_copy(acc.at[pl.ds(row0, ROWS_PER_SUB)],
                  out_hbm.at[pl.ds(row0, ROWS_PER_SUB), c])


_sc_scatter = pl.kernel(
    _sc_body2,
    out_type=jax.ShapeDtypeStruct((N, 2, HH), jnp.float32),
    mesh=plsc.VectorSubcoreMesh(core_axis_name="c", subcore_axis_name="s"),
    scratch_types=[
        pltpu.VMEM((CHUNK,), jnp.int32),
        pltpu.VMEM((CHUNK,), jnp.int32),
        pltpu.VMEM((CHUNK,), jnp.int32),
        pltpu.VMEM((CHUNK, HH), jnp.float32),
        pltpu.VMEM((CHUNK,), jnp.int32),
        pltpu.VMEM((CHUNK,), jnp.int32),
        pltpu.VMEM((CHUNK,), jnp.int32),
        pltpu.VMEM((CHUNK, HH), jnp.float32),
        pltpu.VMEM_SHARED((ACC_ROWS, HH), jnp.float32),
        pltpu.SemaphoreType.DMA,
        pltpu.SemaphoreType.DMA,
        pltpu.SemaphoreType.DMA,
    ],
)

_BLK = 1000


def _enc_body(x_ref, w_ref, b_ref, o_ref):
  o_ref[...] = (
      jnp.dot(x_ref[...], w_ref[...], preferred_element_type=jnp.float32)
      + b_ref[...])


_enc = pl.pallas_call(
    _enc_body,
    grid=(N // _BLK,),
    in_specs=[
        pl.BlockSpec((_BLK, D), lambda i: (i, 0)),
        pl.BlockSpec((D, H), lambda i: (0, 0)),
        pl.BlockSpec((1, H), lambda i: (0, 0)),
    ],
    out_specs=pl.BlockSpec((_BLK, H), lambda i: (i, 0)),
    out_shape=jax.ShapeDtypeStruct((N, H), jnp.float32),
)


def _mlp_body(s_ref, w1_ref, b1_ref, w2_ref, b2_ref, o_ref, *, elu):
  t = (jnp.dot(s_ref[...], w1_ref[...], preferred_element_type=jnp.float32)
       + b1_ref[...])
  t = jnp.maximum(t, 0.0)
  y = (jnp.dot(t, w2_ref[...], preferred_element_type=jnp.float32)
       + b2_ref[...])
  if elu:
    y = jnp.where(y > 0, y, jnp.exp(jnp.minimum(y, 0.0)) - 1.0)
  o_ref[...] = y


def _make_mlp(elu):
  return pl.pallas_call(
      functools.partial(_mlp_body, elu=elu),
      grid=(N // _BLK,),
      in_specs=[
          pl.BlockSpec((_BLK, H), lambda i: (i, 0)),
          pl.BlockSpec((H, H), lambda i: (0, 0)),
          pl.BlockSpec((1, H), lambda i: (0, 0)),
          pl.BlockSpec((H, H), lambda i: (0, 0)),
          pl.BlockSpec((1, H), lambda i: (0, 0)),
      ],
      out_specs=pl.BlockSpec((_BLK, H), lambda i: (i, 0)),
      out_shape=jax.ShapeDtypeStruct((N, H), jnp.float32),
  )


_mlp_elu = _make_mlp(True)
_mlp_plain = _make_mlp(False)


@jax.jit
def kernel(x, edge_index, W_enc, b_enc, W1_0, b1_0, W2_0, b2_0,
           W1_1, b1_1, W2_1, b2_1):
  src = edge_index[0]
  dst = edge_index[1]
  srcp = jnp.concatenate([src, jnp.zeros((E_PAD - E,), jnp.int32)])
  dstp = jnp.concatenate([dst, jnp.full((E_PAD - E,), N, jnp.int32)])

  def gin(h, W1, b1, W2, b2, mlp):
    s = _sc_scatter(h.reshape(2 * N, HH), h.reshape(N, 2, HH), srcp, dstp)
    return mlp(s.reshape(N, H), W1, b1.reshape(1, H), W2, b2.reshape(1, H))

  h = _enc(x, W_enc, b_enc.reshape(1, H))
  h = gin(h, W1_0, b1_0, W2_0, b2_0, _mlp_elu)
  h = gin(h, W1_1, b1_1, W2_1, b2_1, _mlp_plain)
  return h


# speedup vs baseline: 1.3928x; 1.1412x over previous
import functools

import jax
import jax.numpy as jnp
from jax import lax
from jax.experimental import pallas as pl
from jax.experimental.pallas import tpu as pltpu
from jax.experimental.pallas import tpu_sc as plsc

N = 10000
E = 320000
D = 128
H = 256
HH = 128
NSUB = 16
CHUNK = 128
NCHUNK = 158
EPW = NCHUNK * CHUNK
E_PAD = EPW * NSUB
ACC_ROWS = 10016
STRIPE = 624


def _sc_body2(hL_hbm, hR_hbm, src_hbm, dst_hbm, outL_hbm, outR_hbm,
              srcbuf_a, dstbuf_a, rows_a,
              srcbuf_b, dstbuf_b, rows_b,
              acc, sem_i, sem_a, sem_b):
  c = lax.axis_index("c")
  s = lax.axis_index("s")
  row0 = s * STRIPE
  ebase = s * EPW

  def striped_copy(src_at, dst_at):
    @pl.when(s < NSUB - 1)
    def _():
      pltpu.sync_copy(src_at(row0, STRIPE), dst_at(row0, STRIPE))

    @pl.when(s == NSUB - 1)
    def _():
      pltpu.sync_copy(src_at(row0, N - (NSUB - 1) * STRIPE),
                      dst_at(row0, N - (NSUB - 1) * STRIPE))

  def run(table_hbm, out_hbm):
    striped_copy(lambda r, n: table_hbm.at[pl.ds(r, n)],
                 lambda r, n: acc.at[pl.ds(r, n)])
    plsc.subcore_barrier()

    def prep(i, srcbuf, dstbuf, rows, sem):
      off = ebase + i * CHUNK
      pltpu.sync_copy(src_hbm.at[pl.ds(off, CHUNK)], srcbuf)
      pltpu.sync_copy(dst_hbm.at[pl.ds(off, CHUNK)], dstbuf)
      pltpu.async_copy(table_hbm.at[srcbuf], rows, sem)

    def finish(srcbuf, dstbuf, rows, sem):
      pltpu.make_async_copy(table_hbm.at[srcbuf], rows, sem).wait()
      pltpu.sync_copy(rows, acc.at[dstbuf], add=True)

    prep(0, srcbuf_a, dstbuf_a, rows_a, sem_a)

    def pair(jj, carry):
      j0 = jj * 2
      prep(j0 + 1, srcbuf_b, dstbuf_b, rows_b, sem_b)
      finish(srcbuf_a, dstbuf_a, rows_a, sem_a)
      prep(j0 + 2, srcbuf_a, dstbuf_a, rows_a, sem_a)
      finish(srcbuf_b, dstbuf_b, rows_b, sem_b)
      return carry

    lax.fori_loop(0, NCHUNK // 2 - 1, pair, 0)
    prep(NCHUNK - 1, srcbuf_b, dstbuf_b, rows_b, sem_b)
    finish(srcbuf_a, dstbuf_a, rows_a, sem_a)
    finish(srcbuf_b, dstbuf_b, rows_b, sem_b)

    plsc.subcore_barrier()
    striped_copy(lambda r, n: acc.at[pl.ds(r, n)],
                 lambda r, n: out_hbm.at[pl.ds(r, n)])

  @pl.when(c == 0)
  def _():
    run(hL_hbm, outL_hbm)

  @pl.when(c == 1)
  def _():
    run(hR_hbm, outR_hbm)


_sc_scatter = pl.kernel(
    _sc_body2,
    out_type=(jax.ShapeDtypeStruct((N, HH), jnp.float32),
              jax.ShapeDtypeStruct((N, HH), jnp.float32)),
    mesh=plsc.VectorSubcoreMesh(core_axis_name="c", subcore_axis_name="s"),
    scratch_types=[
        pltpu.VMEM((CHUNK,), jnp.int32),
        pltpu.VMEM((CHUNK,), jnp.int32),
        pltpu.VMEM((CHUNK, HH), jnp.float32),
        pltpu.VMEM((CHUNK,), jnp.int32),
        pltpu.VMEM((CHUNK,), jnp.int32),
        pltpu.VMEM((CHUNK, HH), jnp.float32),
        pltpu.VMEM_SHARED((ACC_ROWS, HH), jnp.float32),
        pltpu.SemaphoreType.DMA,
        pltpu.SemaphoreType.DMA,
        pltpu.SemaphoreType.DMA,
    ],
)

_BLK = 1000


def _enc_body(x_ref, w_ref, b_ref, oL_ref, oR_ref):
  y = (jnp.dot(x_ref[...], w_ref[...], preferred_element_type=jnp.float32)
       + b_ref[...])
  oL_ref[...] = y[:, :HH]
  oR_ref[...] = y[:, HH:]


_enc = pl.pallas_call(
    _enc_body,
    grid=(N // _BLK,),
    in_specs=[
        pl.BlockSpec((_BLK, D), lambda i: (i, 0)),
        pl.BlockSpec((D, H), lambda i: (0, 0)),
        pl.BlockSpec((1, H), lambda i: (0, 0)),
    ],
    out_specs=[
        pl.BlockSpec((_BLK, HH), lambda i: (i, 0)),
        pl.BlockSpec((_BLK, HH), lambda i: (i, 0)),
    ],
    out_shape=(jax.ShapeDtypeStruct((N, HH), jnp.float32),
               jax.ShapeDtypeStruct((N, HH), jnp.float32)),
)


def _mlp_body(sL_ref, sR_ref, w1_ref, b1_ref, w2_ref, b2_ref, *out_refs,
              elu, split):
  x = jnp.concatenate([sL_ref[...], sR_ref[...]], axis=1)
  t = (jnp.dot(x, w1_ref[...], preferred_element_type=jnp.float32)
       + b1_ref[...])
  t = jnp.maximum(t, 0.0)
  y = (jnp.dot(t, w2_ref[...], preferred_element_type=jnp.float32)
       + b2_ref[...])
  if elu:
    y = jnp.where(y > 0, y, jnp.exp(jnp.minimum(y, 0.0)) - 1.0)
  if split:
    out_refs[0][...] = y[:, :HH]
    out_refs[1][...] = y[:, HH:]
  else:
    out_refs[0][...] = y


def _make_mlp(elu, split):
  if split:
    out_specs = [pl.BlockSpec((_BLK, HH), lambda i: (i, 0)),
                 pl.BlockSpec((_BLK, HH), lambda i: (i, 0))]
    out_shape = (jax.ShapeDtypeStruct((N, HH), jnp.float32),
                 jax.ShapeDtypeStruct((N, HH), jnp.float32))
  else:
    out_specs = pl.BlockSpec((_BLK, H), lambda i: (i, 0))
    out_shape = jax.ShapeDtypeStruct((N, H), jnp.float32)
  return pl.pallas_call(
      functools.partial(_mlp_body, elu=elu, split=split),
      grid=(N // _BLK,),
      in_specs=[
          pl.BlockSpec((_BLK, HH), lambda i: (i, 0)),
          pl.BlockSpec((_BLK, HH), lambda i: (i, 0)),
          pl.BlockSpec((H, H), lambda i: (0, 0)),
          pl.BlockSpec((1, H), lambda i: (0, 0)),
          pl.BlockSpec((H, H), lambda i: (0, 0)),
          pl.BlockSpec((1, H), lambda i: (0, 0)),
      ],
      out_specs=out_specs,
      out_shape=out_shape,
  )


_mlp_elu_split = _make_mlp(True, True)
_mlp_final = _make_mlp(False, False)


@jax.jit
def kernel(x, edge_index, W_enc, b_enc, W1_0, b1_0, W2_0, b2_0,
           W1_1, b1_1, W2_1, b2_1):
  src = edge_index[0]
  dst = edge_index[1]
  srcp = jnp.concatenate([src, jnp.zeros((E_PAD - E,), jnp.int32)])
  dstp = jnp.concatenate([dst, jnp.full((E_PAD - E,), N, jnp.int32)])

  def gin(hL, hR, W1, b1, W2, b2, mlp):
    sL, sR = _sc_scatter(hL, hR, srcp, dstp)
    return mlp(sL, sR, W1, b1.reshape(1, H), W2, b2.reshape(1, H))

  hL, hR = _enc(x, W_enc, b_enc.reshape(1, H))
  hL, hR = gin(hL, hR, W1_0, b1_0, W2_0, b2_0, _mlp_elu_split)
  return gin(hL, hR, W1_1, b1_1, W2_1, b2_1, _mlp_final)
